# baseline (device time: 201668 ns/iter reference)
import jax
import jax.numpy as jnp
from jax import lax
from jax.experimental import pallas as pl
from jax.experimental.pallas import tpu as pltpu

P = 8
MB = 1024
KB = 1024
G = P // 2
N = 4096
NHDIV = 4
NH = N // NHDIV

_DEV_TYPE = getattr(pl, "DeviceIdType", None) or pltpu.DeviceIdType


def kernel(x, w_mat):
    xb = x.astype(jnp.bfloat16)

    def body(x_ref, w_ref, out_ref, gather, wbuf,
             send_sems, recv_sems, local_sem, w_sems):
        g = pl.program_id(0)
        h = pl.program_id(1)
        s = g * NHDIV + h
        my = lax.axis_index("i")

        def w_copies(slot, pair, half):
            c0 = (my + 2 * pair) % P
            c1 = (my + 2 * pair + 1) % P
            return [
                pltpu.make_async_copy(
                    w_ref.at[pl.ds(c0 * KB, KB), pl.ds(half * NH, NH)],
                    wbuf.at[slot, pl.ds(0, KB), :],
                    w_sems.at[slot],
                ),
                pltpu.make_async_copy(
                    w_ref.at[pl.ds(c1 * KB, KB), pl.ds(half * NH, NH)],
                    wbuf.at[slot, pl.ds(KB, KB), :],
                    w_sems.at[slot],
                ),
            ]

        def local_copy():
            return pltpu.make_async_copy(
                x_ref.at[pl.ds(my * MB, MB), :],
                gather.at[0, :, pl.ds(0, KB)],
                local_sem,
            )

        def rdma_to(peer):
            c = (my - peer) % P
            return pltpu.make_async_remote_copy(
                src_ref=x_ref.at[pl.ds(peer * MB, MB), :],
                dst_ref=gather.at[c // 2, :, pl.ds((c % 2) * KB, KB)],
                send_sem=send_sems.at[peer],
                recv_sem=recv_sems.at[my],
                device_id=(peer,),
                device_id_type=_DEV_TYPE.MESH,
            )

        def wait_recv_from(src):
            pltpu.make_async_remote_copy(
                src_ref=x_ref.at[pl.ds(0, MB), :],
                dst_ref=gather.at[0, :, pl.ds(0, KB)],
                send_sem=send_sems.at[src],
                recv_sem=recv_sems.at[src],
                device_id=(my,),
                device_id_type=_DEV_TYPE.MESH,
            ).wait_recv()

        @pl.when(s == 0)
        def _():
            bsem = pltpu.get_barrier_semaphore()
            for k in range(1, P):
                pl.semaphore_signal(
                    bsem, inc=1,
                    device_id=((my + k) % P,),
                    device_id_type=_DEV_TYPE.MESH,
                )
            pl.semaphore_wait(bsem, P - 1)

            local_copy().start()
            for k in range(1, P):
                rdma_to((my - k) % P).start()
            for cp in w_copies(0, 0, 0):
                cp.start()

        @pl.when(s + 1 < G * NHDIV)
        def _():
            nxt = s + 1
            for cp in w_copies(nxt % 2, nxt // NHDIV, nxt % NHDIV):
                cp.start()

        @pl.when(jnp.logical_and(h == 0, g == 0))
        def _():
            local_copy().wait()
            wait_recv_from((my + 1) % P)

        @pl.when(jnp.logical_and(h == 0, g > 0))
        def _():
            wait_recv_from((my + 2 * g) % P)
            wait_recv_from((my + 2 * g + 1) % P)

        for cp in w_copies(s % 2, g, h):
            cp.wait()

        partial = jnp.dot(
            gather[g],
            wbuf[s % 2].astype(jnp.bfloat16),
            preferred_element_type=jnp.float32,
        )
        nsl = pl.ds(h * NH, NH)

        @pl.when(g == 0)
        def _():
            out_ref[:, nsl] = partial

        @pl.when(g > 0)
        def _():
            out_ref[:, nsl] += partial

        @pl.when(s == G * NHDIV - 1)
        def _():
            for k in range(1, P):
                rdma_to((my + k) % P).wait_send()
            y = out_ref[...]
            cg = 0.7978845608028654
            out_ref[...] = 0.5 * y * (1.0 + jnp.tanh(cg * (y + 0.044715 * y * y * y)))

    return pl.pallas_call(
        body,
        grid=(G, NHDIV),
        out_shape=jax.ShapeDtypeStruct((MB, N), jnp.float32),
        in_specs=[
            pl.BlockSpec(memory_space=pl.ANY),
            pl.BlockSpec(memory_space=pl.ANY),
        ],
        out_specs=pl.BlockSpec((MB, N), lambda g, h: (0, 0)),
        scratch_shapes=[
            pltpu.VMEM((G, MB, 2 * KB), jnp.bfloat16),
            pltpu.VMEM((2, 2 * KB, NH), jnp.float32),
            pltpu.SemaphoreType.DMA((P,)),
            pltpu.SemaphoreType.DMA((P,)),
            pltpu.SemaphoreType.DMA,
            pltpu.SemaphoreType.DMA((2,)),
        ],
        compiler_params=pltpu.CompilerParams(
            collective_id=0, vmem_limit_bytes=100 * 1024 * 1024,
        ),
    )(xb, w_mat)
